# baseline (speedup 1.0000x reference)
import functools
import jax
import jax.numpy as jnp
from jax import lax
from jax.experimental import pallas as pl
from jax.experimental.pallas import tpu as pltpu
from jax.experimental.pallas import tpu_sc as plsc

_W = 128
_C = 96
_D = 48
_CCHUNKS = _C // 16
_PAD = _D


def _body(l_hbm, r_hbm, out_hbm, l_v, rpad_v, out_v, sem):
    n_cores = 2
    n_sub = 16
    wid = lax.axis_index("s") * n_cores + lax.axis_index("c")
    n_workers = n_cores * n_sub
    nrows = l_hbm.shape[0]
    rows_per = nrows // n_workers

    zero = jnp.zeros((16,), jnp.float32)

    def zero_chunk(i, _):
        rpad_v[pl.ds(i * 16, 16)] = zero
        return 0

    lax.fori_loop(0, _PAD * _C // 16, zero_chunk, 0)

    riota = lax.iota(jnp.int32, 16) * _C

    def do_row(r, _):
        row = wid * rows_per + r
        pltpu.sync_copy(l_hbm.at[row], l_v)
        pltpu.sync_copy(r_hbm.at[row], rpad_v.at[pl.ds(_PAD * _C, _W * _C)])

        def do_w(w, _):
            bases = [
                jnp.full((16,), (_PAD + w - 16 * dv) * _C, jnp.int32) - riota
                for dv in range(_D // 16)
            ]
            accs = [jnp.zeros((16,), jnp.float32) for _ in range(_D // 16)]
            for cb in range(_CCHUNKS):
                lvec = l_v[w, pl.ds(16 * cb, 16)]
                for cc in range(16):
                    c = 16 * cb + cc
                    lb = jnp.full((16,), lvec[cc], jnp.float32)
                    for dv in range(_D // 16):
                        vals = plsc.load_gather(rpad_v, [bases[dv] + c])
                        accs[dv] = accs[dv] + lb * vals
            for dv in range(_D // 16):
                out_v[w, pl.ds(16 * dv, 16)] = accs[dv] * (1.0 / _C)
            return 0

        lax.fori_loop(0, _W, do_w, 0)
        pltpu.sync_copy(out_v, out_hbm.at[row])
        return 0

    lax.fori_loop(0, rows_per, do_row, 0)


def kernel(left_features, right_features):
    b, h, w, c = left_features.shape
    l2 = left_features.reshape(b * h, w, c)
    r2 = right_features.reshape(b * h, w * c)
    mesh = plsc.VectorSubcoreMesh(
        core_axis_name="c", subcore_axis_name="s", num_cores=2, num_subcores=16
    )
    out = pl.kernel(
        _body,
        out_type=jax.ShapeDtypeStruct((b * h, w, _D), jnp.float32),
        mesh=mesh,
        compiler_params=pltpu.CompilerParams(needs_layout_passes=False),
        scratch_types=[
            pltpu.VMEM((_W, _C), jnp.float32),
            pltpu.VMEM(((_PAD + _W) * _C,), jnp.float32),
            pltpu.VMEM((_W, _D), jnp.float32),
            pltpu.SemaphoreType.DMA,
        ],
    )(l2, r2)
    return out.reshape(b, h, w, _D)


# speedup vs baseline: 9.0681x; 9.0681x over previous
import functools
import jax
import jax.numpy as jnp
from jax import lax
from jax.experimental import pallas as pl
from jax.experimental.pallas import tpu as pltpu
from jax.experimental.pallas import tpu_sc as plsc

_W = 128
_C = 96
_D = 48
_CCHUNKS = _C // 16
_PAD = _D
_NW = 8
_NK = 4
_NKB = (_D + _NK - 1) // _NK + 1


def _valid_pairs(k):
    return [
        (i, j)
        for i in range(_NW)
        for j in range(_NK)
        if 0 <= _D + i - _NK * k - j < _D
    ]


def _body(l_hbm, r_hbm, out_hbm, l_v, rpad_v, out_v, sem):
    n_cores = 2
    n_sub = 16
    wid = lax.axis_index("s") * n_cores + lax.axis_index("c")
    n_workers = n_cores * n_sub
    nrows = l_hbm.shape[0]
    rows_per = nrows // n_workers

    zero16 = jnp.zeros((16,), jnp.float32)
    scale = jnp.float32(1.0 / _C)
    lane_mask = lax.iota(jnp.int32, 16) == 15

    def zero_chunk(i, _):
        rpad_v[pl.ds(i * 16, 16)] = zero16
        return 0

    lax.fori_loop(0, _PAD * _C // 16, zero_chunk, 0)

    def emit_tile(w0, k, valid):
        used_i = sorted({i for i, _ in valid})
        used_j = sorted({j for _, j in valid})
        accs = {p: zero16 for p in valid}
        for cb in range(_CCHUNKS):
            lv = {i: l_v[w0 + i, pl.ds(16 * cb, 16)] for i in used_i}
            rv = {
                j: rpad_v[pl.ds((w0 + _NK * k + j) * _C + 16 * cb, 16)]
                for j in used_j
            }
            for (i, j) in valid:
                accs[(i, j)] = accs[(i, j)] + lv[i] * rv[j]
        for (i, j) in valid:
            s = plsc.cumsum(accs[(i, j)] * scale)
            tgt = w0 * _D + _D + (_D + 1) * i - _NK * k - j
            plsc.store_scatter(
                out_v, [jnp.full((16,), tgt, jnp.int32)], s, mask=lane_mask
            )

    def do_row(r, _):
        row = wid * rows_per + r
        pltpu.sync_copy(l_hbm.at[row], l_v)
        pltpu.sync_copy(r_hbm.at[row], rpad_v.at[pl.ds(_PAD * _C, _W * _C)])

        def do_wblock(wb, _):
            w0 = wb * _NW
            emit_tile(w0, 0, _valid_pairs(0))
            emit_tile(w0, 1, _valid_pairs(1))

            def interior(k, _):
                emit_tile(w0, k, [(i, j) for i in range(_NW) for j in range(_NK)])
                return 0

            lax.fori_loop(2, _NKB - 1, interior, 0)
            emit_tile(w0, _NKB - 1, _valid_pairs(_NKB - 1))
            emit_tile(w0, _NKB, _valid_pairs(_NKB))
            return 0

        lax.fori_loop(0, _W // _NW, do_wblock, 0)
        pltpu.sync_copy(out_v, out_hbm.at[row])
        return 0

    lax.fori_loop(0, rows_per, do_row, 0)


def kernel(left_features, right_features):
    b, h, w, c = left_features.shape
    l2 = left_features.reshape(b * h, w, c)
    r2 = right_features.reshape(b * h, w * c)
    mesh = plsc.VectorSubcoreMesh(
        core_axis_name="c", subcore_axis_name="s", num_cores=2, num_subcores=16
    )
    out = pl.kernel(
        _body,
        out_type=jax.ShapeDtypeStruct((b * h, w * _D), jnp.float32),
        mesh=mesh,
        compiler_params=pltpu.CompilerParams(needs_layout_passes=False),
        scratch_types=[
            pltpu.VMEM((_W, _C), jnp.float32),
            pltpu.VMEM(((_PAD + _W) * _C,), jnp.float32),
            pltpu.VMEM((_W * _D,), jnp.float32),
            pltpu.SemaphoreType.DMA,
        ],
    )(l2, r2)
    return out.reshape(b, h, w, _D)
